# Initial kernel scaffold; baseline (speedup 1.0000x reference)
import jax
import jax.numpy as jnp
from jax.experimental import pallas as pl


def kernel(x, edge_index, W1, b1, W2, b2):
    raise NotImplementedError("write your pallas kernel here")


# speedup vs baseline: 9.4576x; 9.4576x over previous
import functools

import jax
import jax.numpy as jnp
from jax import lax
from jax.experimental import pallas as pl
from jax.experimental.pallas import tpu as pltpu
from jax.experimental.pallas import tpu_sc as plsc

N = 10000
E = 320000
D = 128
NP = 10240

NC = 2
NS = 16
NW = NC * NS
C = 128
NCH = 80
HNCH = 40
EPW = NCH * C
EP = NW * EPW
ROWS_PT = NP // NS

_mesh = plsc.VectorSubcoreMesh(core_axis_name="c", subcore_axis_name="s")


def _fill_f32(ref, n, value):
    def body(i, _):
        ref[pl.ds(i * 16, 16)] = jnp.full((16,), value, jnp.float32)
        return 0
    lax.fori_loop(0, n // 16, body, 0)


@functools.partial(
    pl.kernel,
    out_type=jax.ShapeDtypeStruct((NC, NP), jnp.float32),
    mesh=_mesh,
    scratch_types=[
        pltpu.VMEM((NCH, C), jnp.int32),
        pltpu.VMEM((C,), jnp.float32),
        pltpu.VMEM((ROWS_PT,), jnp.float32),
        pltpu.VMEM_SHARED((NP,), jnp.float32),
        pltpu.SemaphoreType.DMA,
    ],
)
def _deg_kernel(dst_hbm, out_hbm, idx_v, ones_v, zbuf, acc_sp, sem):
    c = lax.axis_index("c")
    s = lax.axis_index("s")
    wid = c * NS + s
    _fill_f32(ones_v, C, 1.0)
    _fill_f32(zbuf, ROWS_PT, 0.0)
    pltpu.sync_copy(zbuf, acc_sp.at[pl.ds(s * ROWS_PT, ROWS_PT)])
    pltpu.async_copy(dst_hbm.at[wid], idx_v, sem).wait()
    plsc.subcore_barrier()
    def body(j, _):
        for b in range(8):
            pltpu.async_copy(ones_v, acc_sp.at[idx_v.at[j * 8 + b]], sem,
                             add=True)
        for b in range(8):
            pltpu.make_async_copy(ones_v, acc_sp.at[idx_v.at[0]], sem).wait()
        return 0
    lax.fori_loop(0, NCH // 8, body, 0)
    plsc.subcore_barrier()
    pltpu.sync_copy(acc_sp.at[pl.ds(s * ROWS_PT, ROWS_PT)],
                    out_hbm.at[c, pl.ds(s * ROWS_PT, ROWS_PT)])


@functools.partial(
    pl.kernel,
    out_type=jax.ShapeDtypeStruct((NC, NP, D), jnp.float32),
    mesh=_mesh,
    scratch_types=[
        pltpu.VMEM((HNCH, C), jnp.int32),
        pltpu.VMEM((HNCH, C), jnp.int32),
        pltpu.VMEM((C, D), jnp.float32),
        pltpu.VMEM((C, D), jnp.float32),
        pltpu.VMEM_SHARED((NP, D), jnp.float32),
        pltpu.SemaphoreType.DMA,
        pltpu.SemaphoreType.DMA,
        pltpu.SemaphoreType.DMA,
    ],
)
def _agg_kernel(g_hbm, src_hbm, dst_hbm, out_hbm, src_v, dst_v, rows0, rows1,
                acc_sp, sem0, sem1, isem):
    c = lax.axis_index("c")
    s = lax.axis_index("s")
    wid = c * NS + s
    def zrow(r, _):
        def zcol(k, _):
            rows0[r, pl.ds(k * 16, 16)] = jnp.zeros((16,), jnp.float32)
            return 0
        lax.fori_loop(0, D // 16, zcol, 0)
        return 0
    lax.fori_loop(0, C, zrow, 0)
    def zinit(k, _):
        pltpu.sync_copy(rows0, acc_sp.at[pl.ds(s * ROWS_PT + k * C, C)])
        return 0
    lax.fori_loop(0, ROWS_PT // C, zinit, 0)
    plsc.subcore_barrier()
    for h in range(NCH // HNCH):
        pltpu.async_copy(src_hbm.at[wid, pl.ds(h * HNCH, HNCH)], src_v,
                         isem)
        pltpu.async_copy(dst_hbm.at[wid, pl.ds(h * HNCH, HNCH)], dst_v,
                         isem)
        pltpu.make_async_copy(src_hbm.at[wid, pl.ds(0, HNCH)], src_v,
                              isem).wait()
        pltpu.make_async_copy(dst_hbm.at[wid, pl.ds(0, HNCH)], dst_v,
                              isem).wait()
        pltpu.async_copy(g_hbm.at[src_v.at[0]], rows0, sem0)
        def body(j, _):
            @pl.when(j % 2 == 0)
            def _even():
                @pl.when(j + 1 < HNCH)
                def _():
                    pltpu.async_copy(g_hbm.at[src_v.at[j + 1]], rows1, sem1)
                pltpu.make_async_copy(g_hbm.at[src_v.at[0]], rows0,
                                      sem0).wait()
                pltpu.sync_copy(rows0, acc_sp.at[dst_v.at[j]], add=True)
            @pl.when(j % 2 == 1)
            def _odd():
                @pl.when(j + 1 < HNCH)
                def _():
                    pltpu.async_copy(g_hbm.at[src_v.at[j + 1]], rows0, sem0)
                pltpu.make_async_copy(g_hbm.at[src_v.at[0]], rows1,
                                      sem1).wait()
                pltpu.sync_copy(rows1, acc_sp.at[dst_v.at[j]], add=True)
            return 0
        lax.fori_loop(0, HNCH, body, 0)
    plsc.subcore_barrier()
    pltpu.sync_copy(acc_sp.at[pl.ds(s * ROWS_PT, ROWS_PT)],
                    out_hbm.at[c, pl.ds(s * ROWS_PT, ROWS_PT), :])


RB = 2048


def _k1_body(d_ref, x_ref, w_ref, o_ref):
    dis = lax.rsqrt(d_ref[:, 0:1] + d_ref[:, 1:2] + 1.0)
    o_ref[...] = dis * jnp.dot(x_ref[...], w_ref[...],
                               preferred_element_type=jnp.float32)


def _k2_body(d_ref, a_ref, g_ref, b_ref, w_ref, o_ref):
    dis = lax.rsqrt(d_ref[:, 0:1] + d_ref[:, 1:2] + 1.0)
    t = jnp.maximum(dis * (a_ref[0] + a_ref[1] + g_ref[...]) + b_ref[...], 0.0)
    o_ref[...] = dis * jnp.dot(t, w_ref[...],
                               preferred_element_type=jnp.float32)


def _k3_body(d_ref, a_ref, g_ref, b_ref, o_ref):
    dis = lax.rsqrt(d_ref[:, 0:1] + d_ref[:, 1:2] + 1.0)
    o_ref[...] = jnp.maximum(
        dis * (a_ref[0] + a_ref[1] + g_ref[...]) + b_ref[...], 0.0)


_dspec = pl.BlockSpec((RB, 2), lambda i: (i, 0))
_rspec = pl.BlockSpec((RB, D), lambda i: (i, 0))
_aspec = pl.BlockSpec((NC, RB, D), lambda i: (0, i, 0))
_wspec = pl.BlockSpec((D, D), lambda i: (0, 0))
_bspec = pl.BlockSpec((1, D), lambda i: (0, 0))
_grid = (NP // RB,)
_out128 = jax.ShapeDtypeStruct((NP, D), jnp.float32)

_k1 = pl.pallas_call(_k1_body, grid=_grid, out_shape=_out128,
                     in_specs=[_dspec, _rspec, _wspec], out_specs=_rspec)
_k2 = pl.pallas_call(_k2_body, grid=_grid, out_shape=_out128,
                     in_specs=[_dspec, _aspec, _rspec, _bspec, _wspec],
                     out_specs=_rspec)
_k3 = pl.pallas_call(_k3_body, grid=_grid, out_shape=_out128,
                     in_specs=[_dspec, _aspec, _rspec, _bspec],
                     out_specs=_rspec)


def kernel(x, edge_index, W1, b1, W2, b2):
    pad = jnp.full((EP - E,), NP - 1, jnp.int32)
    src = jnp.concatenate([edge_index[0], pad]).reshape(NW, NCH, C)
    dst = jnp.concatenate([edge_index[1], pad]).reshape(NW, NCH, C)
    xp = jnp.concatenate([x, jnp.zeros((NP - N, D), jnp.float32)], axis=0)
    deg = _deg_kernel(dst)
    deg_nt = deg.T
    b1r = b1.reshape(1, D)
    b2r = b2.reshape(1, D)
    g1 = _k1(deg_nt, xp, W1)
    acc1 = _agg_kernel(g1, src, dst)
    g2 = _k2(deg_nt, acc1, g1, b1r, W2)
    acc2 = _agg_kernel(g2, src, dst)
    return _k3(deg_nt, acc2, g2, b2r)[:N]
